# baseline (speedup 1.0000x reference)
import functools

import jax
import jax.numpy as jnp
from jax import lax
from jax.experimental import pallas as pl
from jax.experimental.pallas import tpu as pltpu
from jax.experimental.pallas import tpu_sc as plsc

B = 16384
D = 32
NROWS = 1000000
NC = 2
NS = 16
NW = NC * NS
BPW = B // NW
EPW = BPW * D
NCH = 4

_sc_mesh = plsc.VectorSubcoreMesh(core_axis_name="c", subcore_axis_name="s")


@functools.partial(
    pl.kernel,
    mesh=_sc_mesh,
    out_type=[
        jax.ShapeDtypeStruct((B * D,), jnp.float32),
        jax.ShapeDtypeStruct((B * D,), jnp.float32),
    ],
    scratch_types=[
        pltpu.VMEM((BPW,), jnp.int32),
        pltpu.VMEM((BPW,), jnp.int32),
        pltpu.VMEM((EPW,), jnp.int32),
        pltpu.VMEM((EPW,), jnp.int32),
        pltpu.VMEM((EPW,), jnp.float32),
        pltpu.VMEM((EPW,), jnp.float32),
        pltpu.SemaphoreType.DMA,
        pltpu.SemaphoreType.DMA,
    ],
)
def _sc_gather(uid_hbm, iid_hbm, utab_hbm, itab_hbm, uout_hbm, iout_hbm,
               idu_v, idi_v, exu_v, exi_v, stu_v, sti_v, usem, isem):
    wid = lax.axis_index("s") * NC + lax.axis_index("c")
    base = wid * BPW
    pltpu.sync_copy(uid_hbm.at[pl.ds(base, BPW)], idu_v)
    pltpu.sync_copy(iid_hbm.at[pl.ds(base, BPW)], idi_v)
    lane = lax.iota(jnp.int32, 16)

    def chunk_body(ch, _):
        for g in range(BPW // NCH // 16):
            off = ch * (BPW // NCH) + g * 16
            ubase = idu_v[pl.ds(off, 16)] * D
            ibase = idi_v[pl.ds(off, 16)] * D
            for l in range(16):
                p = (off + l) * D
                bu = ubase[l]
                bi = ibase[l]
                exu_v[pl.ds(p, 16)] = bu + lane
                exu_v[pl.ds(p + 16, 16)] = (bu + 16) + lane
                exi_v[pl.ds(p, 16)] = bi + lane
                exi_v[pl.ds(p + 16, 16)] = (bi + 16) + lane
        return _

    lax.fori_loop(0, NCH, chunk_body, None)
    cu = pltpu.async_copy(utab_hbm.at[exu_v], stu_v, usem)
    ci = pltpu.async_copy(itab_hbm.at[exi_v], sti_v, isem)
    cu.wait()
    ci.wait()
    pltpu.sync_copy(stu_v, uout_hbm.at[pl.ds(base * D, EPW)])
    pltpu.sync_copy(sti_v, iout_hbm.at[pl.ds(base * D, EPW)])


BLK = 1024


def _mlp_body(xu_ref, xv_ref, w1a_ref, w1b_ref, b1_ref, w2_ref, b2_ref,
              w3_ref, b3_ref, out_ref):
    h = jnp.dot(xu_ref[...], w1a_ref[...], preferred_element_type=jnp.float32)
    h = h + jnp.dot(xv_ref[...], w1b_ref[...], preferred_element_type=jnp.float32)
    h = jnp.maximum(h + b1_ref[...], 0.0)
    h2 = jnp.dot(h, w2_ref[...], preferred_element_type=jnp.float32)
    h2 = jnp.maximum(h2 + b2_ref[...], 0.0)
    out_ref[...] = jnp.sum(h2 * w3_ref[...], axis=1, keepdims=True) + b3_ref[...]


_mlp = pl.pallas_call(
    _mlp_body,
    grid=(B // BLK,),
    in_specs=[
        pl.BlockSpec((BLK, D), lambda i: (i, 0)),
        pl.BlockSpec((BLK, D), lambda i: (i, 0)),
        pl.BlockSpec((D, 64), lambda i: (0, 0)),
        pl.BlockSpec((D, 64), lambda i: (0, 0)),
        pl.BlockSpec((1, 64), lambda i: (0, 0)),
        pl.BlockSpec((64, 32), lambda i: (0, 0)),
        pl.BlockSpec((1, 32), lambda i: (0, 0)),
        pl.BlockSpec((1, 32), lambda i: (0, 0)),
        pl.BlockSpec((1, 1), lambda i: (0, 0)),
    ],
    out_specs=pl.BlockSpec((BLK, 1), lambda i: (i, 0)),
    out_shape=jax.ShapeDtypeStruct((B, 1), jnp.float32),
)


def kernel(user_ids, item_ids, user_table, item_table, W1, b1, W2, b2, W3, b3):
    uid = user_ids.astype(jnp.int32)
    iid = item_ids.astype(jnp.int32)
    uflat, iflat = _sc_gather(uid, iid, user_table.reshape(NROWS * D),
                              item_table.reshape(NROWS * D))
    urows = uflat.reshape(B, D)
    irows = iflat.reshape(B, D)
    out = _mlp(urows, irows, W1[:D], W1[D:], b1.reshape(1, 64), W2,
               b2.reshape(1, 32), W3.reshape(1, 32), b3.reshape(1, 1))
    return out[:, 0]
